# baseline (speedup 1.0000x reference)
import functools

import jax
import jax.numpy as jnp
from jax import lax
from jax.experimental import pallas as pl
from jax.experimental.pallas import tpu as pltpu
from jax.experimental.pallas import tpu_sc as plsc

N_TGT = 50000
N_SAMP = 10
D = 128
LANES = 16
NW = 32
NSUB = 16
T_CHUNK = 40
ROWS_CHUNK = T_CHUNK * N_SAMP
G_SIZE = 80
N_GRP = ROWS_CHUNK // G_SIZE
LAST_BASE = N_TGT - T_CHUNK
K_PER_W = (-(-(-(-N_TGT // T_CHUNK)) // NW)) | 1


def _mean_agg(neigh_flat, table, scale16):
    mesh = plsc.VectorSubcoreMesh(core_axis_name="c", subcore_axis_name="s")

    @functools.partial(
        pl.kernel,
        mesh=mesh,
        out_type=jax.ShapeDtypeStruct((N_TGT, D), jnp.float32),
        scratch_types=[
            pltpu.VMEM((K_PER_W * ROWS_CHUNK,), jnp.int32),
            pltpu.VMEM((2, ROWS_CHUNK, D), jnp.float32),
            pltpu.VMEM((2, T_CHUNK, D), jnp.float32),
            pltpu.VMEM((LANES,), jnp.float32),
            pltpu.SemaphoreType.DMA,
            pltpu.SemaphoreType.DMA,
            pltpu.SemaphoreType.DMA,
            pltpu.SemaphoreType.DMA,
            pltpu.SemaphoreType.DMA,
        ],
    )
    def k(neigh_hbm, table_hbm, scale_hbm, out_hbm, idx_all, rows_v, out_v,
          scale_v, sem_i, sem_g0, sem_g1, sem_o0, sem_o1):
        wid = lax.axis_index("s") * 2 + lax.axis_index("c")
        sem_g = (sem_g0, sem_g1)
        sem_o = (sem_o0, sem_o1)

        pltpu.sync_copy(scale_hbm, scale_v)
        scale = scale_v[...]

        def chunk_base(kk):
            c = kk * NW + wid
            return jnp.minimum(c * T_CHUNK, LAST_BASE)

        descs = []
        for kk in range(K_PER_W):
            src = neigh_hbm.at[pl.ds(chunk_base(kk) * N_SAMP, ROWS_CHUNK)]
            descs.append(pltpu.async_copy(
                src, idx_all.at[pl.ds(kk * ROWS_CHUNK, ROWS_CHUNK)], sem_i))
        for dsc in descs:
            dsc.wait()

        def gathers(kk, b):
            return [
                pltpu.make_async_copy(
                    table_hbm.at[
                        idx_all.at[pl.ds(kk * ROWS_CHUNK + j * G_SIZE,
                                         G_SIZE)]],
                    rows_v.at[b, pl.ds(j * G_SIZE, G_SIZE)],
                    sem_g[b])
                for j in range(N_GRP)
            ]

        def fire_g(kk, b):
            for dsc in gathers(kk, b):
                dsc.start()

        def wait_g(kk, b):
            for dsc in gathers(kk, b):
                dsc.wait()

        def out_desc(kk, b):
            return pltpu.make_async_copy(
                out_v.at[b], out_hbm.at[pl.ds(chunk_base(kk), T_CHUNK)],
                sem_o[b])

        def compute(kk, b):
            def t_body(i2, tc):
                for u in range(2):
                    t = i2 * 2 + u
                    r0 = t * N_SAMP
                    for g in range(D // LANES):
                        sl = pl.ds(g * LANES, LANES)
                        vs = [rows_v[b, r0 + s2, sl] for s2 in range(N_SAMP)]
                        while len(vs) > 1:
                            nxt = [vs[i] + vs[i + 1]
                                   for i in range(0, len(vs) - 1, 2)]
                            if len(vs) % 2:
                                nxt.append(vs[-1])
                            vs = nxt
                        out_v[b, t, sl] = vs[0] * scale
                return tc

            lax.fori_loop(0, T_CHUNK // 2, t_body, 0)

        fire_g(0, 0)

        def body(i, carry):
            kk = 2 * i
            fire_g(kk + 1, 1)
            wait_g(kk, 0)

            @pl.when(i > 0)
            def _():
                out_desc(kk - 2, 0).wait()

            compute(kk, 0)
            out_desc(kk, 0).start()
            fire_g(kk + 2, 0)
            wait_g(kk + 1, 1)

            @pl.when(i > 0)
            def _():
                out_desc(kk - 1, 1).wait()

            compute(kk + 1, 1)
            out_desc(kk + 1, 1).start()
            return carry

        lax.fori_loop(0, (K_PER_W - 1) // 2, body, 0)

        last = K_PER_W - 1
        wait_g(last, 0)
        out_desc(last - 2, 0).wait()
        compute(last, 0)
        out_desc(last, 0).start()
        out_desc(last, 0).wait()
        out_desc(last - 1, 1).wait()

    return k(neigh_flat, table, scale16)


def kernel(nodes, to_neighs, table, num_sample):
    del nodes
    neigh_flat = to_neighs.reshape(-1).astype(jnp.int32)
    ns = jnp.minimum(jnp.asarray(num_sample, jnp.float32),
                     jnp.float32(N_SAMP))
    scale16 = jnp.full((LANES,), 1.0, jnp.float32) / ns
    return _mean_agg(neigh_flat, table, scale16)


# speedup vs baseline: 1.0766x; 1.0766x over previous
import functools

import jax
import jax.numpy as jnp
from jax import lax
from jax.experimental import pallas as pl
from jax.experimental.pallas import tpu as pltpu
from jax.experimental.pallas import tpu_sc as plsc

N_TGT = 50000
N_SAMP = 10
D = 128
LANES = 16
NW = 32
NSUB = 16
T_CHUNK = 32
ROWS_CHUNK = T_CHUNK * N_SAMP
G_SIZE = 80
N_GRP = ROWS_CHUNK // G_SIZE
LAST_BASE = N_TGT - T_CHUNK
K_PER_W = (-(-(-(-N_TGT // T_CHUNK)) // NW)) | 1


def _mean_agg(neigh_flat, table, scale16):
    mesh = plsc.VectorSubcoreMesh(core_axis_name="c", subcore_axis_name="s")

    @functools.partial(
        pl.kernel,
        mesh=mesh,
        out_type=jax.ShapeDtypeStruct((N_TGT, D), jnp.float32),
        scratch_types=[
            pltpu.VMEM((K_PER_W * ROWS_CHUNK,), jnp.int32),
            pltpu.VMEM((2, ROWS_CHUNK, D), jnp.float32),
            pltpu.VMEM((2, T_CHUNK, D), jnp.float32),
            pltpu.VMEM((LANES,), jnp.float32),
            pltpu.SemaphoreType.DMA,
            pltpu.SemaphoreType.DMA,
            pltpu.SemaphoreType.DMA,
            pltpu.SemaphoreType.DMA,
            pltpu.SemaphoreType.DMA,
        ],
    )
    def k(neigh_hbm, table_hbm, scale_hbm, out_hbm, idx_all, rows_v, out_v,
          scale_v, sem_i, sem_g0, sem_g1, sem_o0, sem_o1):
        wid = lax.axis_index("s") * 2 + lax.axis_index("c")
        sem_g = (sem_g0, sem_g1)
        sem_o = (sem_o0, sem_o1)

        pltpu.sync_copy(scale_hbm, scale_v)
        scale = scale_v[...]

        def chunk_base(kk):
            c = kk * NW + wid
            return jnp.minimum(c * T_CHUNK, LAST_BASE)

        descs = []
        for kk in range(K_PER_W):
            src = neigh_hbm.at[pl.ds(chunk_base(kk) * N_SAMP, ROWS_CHUNK)]
            descs.append(pltpu.async_copy(
                src, idx_all.at[pl.ds(kk * ROWS_CHUNK, ROWS_CHUNK)], sem_i))
        for dsc in descs:
            dsc.wait()

        def gathers(kk, b):
            return [
                pltpu.make_async_copy(
                    table_hbm.at[
                        idx_all.at[pl.ds(kk * ROWS_CHUNK + j * G_SIZE,
                                         G_SIZE)]],
                    rows_v.at[b, pl.ds(j * G_SIZE, G_SIZE)],
                    sem_g[b])
                for j in range(N_GRP)
            ]

        def fire_g(kk, b):
            for dsc in gathers(kk, b):
                dsc.start()

        def wait_g(kk, b):
            for dsc in gathers(kk, b):
                dsc.wait()

        def out_desc(kk, b):
            return pltpu.make_async_copy(
                out_v.at[b], out_hbm.at[pl.ds(chunk_base(kk), T_CHUNK)],
                sem_o[b])

        def compute(kk, b):
            def t_body(i2, tc):
                for u in range(2):
                    t = i2 * 2 + u
                    r0 = t * N_SAMP
                    for g in range(D // LANES):
                        sl = pl.ds(g * LANES, LANES)
                        vs = [rows_v[b, r0 + s2, sl] for s2 in range(N_SAMP)]
                        while len(vs) > 1:
                            nxt = [vs[i] + vs[i + 1]
                                   for i in range(0, len(vs) - 1, 2)]
                            if len(vs) % 2:
                                nxt.append(vs[-1])
                            vs = nxt
                        out_v[b, t, sl] = vs[0] * scale
                return tc

            lax.fori_loop(0, T_CHUNK // 2, t_body, 0)

        fire_g(0, 0)

        def body(i, carry):
            kk = 2 * i
            fire_g(kk + 1, 1)
            wait_g(kk, 0)

            @pl.when(i > 0)
            def _():
                out_desc(kk - 2, 0).wait()

            compute(kk, 0)
            out_desc(kk, 0).start()
            fire_g(kk + 2, 0)
            wait_g(kk + 1, 1)

            @pl.when(i > 0)
            def _():
                out_desc(kk - 1, 1).wait()

            compute(kk + 1, 1)
            out_desc(kk + 1, 1).start()
            return carry

        lax.fori_loop(0, (K_PER_W - 1) // 2, body, 0)

        last = K_PER_W - 1
        wait_g(last, 0)
        out_desc(last - 2, 0).wait()
        compute(last, 0)
        out_desc(last, 0).start()
        out_desc(last, 0).wait()
        out_desc(last - 1, 1).wait()

    return k(neigh_flat, table, scale16)


def kernel(nodes, to_neighs, table, num_sample):
    del nodes
    neigh_flat = to_neighs.reshape(-1).astype(jnp.int32)
    ns = jnp.minimum(jnp.asarray(num_sample, jnp.float32),
                     jnp.float32(N_SAMP))
    scale16 = jnp.full((LANES,), 1.0, jnp.float32) / ns
    return _mean_agg(neigh_flat, table, scale16)


# speedup vs baseline: 1.2505x; 1.1615x over previous
import functools

import jax
import jax.numpy as jnp
from jax import lax
from jax.experimental import pallas as pl
from jax.experimental.pallas import tpu as pltpu
from jax.experimental.pallas import tpu_sc as plsc

N_TGT = 50000
N_SAMP = 10
D = 128
LANES = 16
NW = 32
T_CHUNK = 32
ROWS_CHUNK = T_CHUNK * N_SAMP
LAST_BASE = N_TGT - T_CHUNK
K_PER_W = (-(-(-(-N_TGT // T_CHUNK)) // NW)) | 1
N_CH = NW * K_PER_W


def _mean_agg(perm_idx, table, scale16):
    mesh = plsc.VectorSubcoreMesh(core_axis_name="c", subcore_axis_name="s")

    @functools.partial(
        pl.kernel,
        mesh=mesh,
        out_type=jax.ShapeDtypeStruct((N_TGT, D), jnp.float32),
        scratch_types=[
            pltpu.VMEM((K_PER_W * ROWS_CHUNK,), jnp.int32),
            pltpu.VMEM((2, T_CHUNK, D), jnp.float32),
            pltpu.VMEM((2, T_CHUNK, D), jnp.float32),
            pltpu.VMEM((LANES,), jnp.float32),
            pltpu.SemaphoreType.DMA,
            pltpu.SemaphoreType.DMA,
            pltpu.SemaphoreType.DMA,
            pltpu.SemaphoreType.DMA,
            pltpu.SemaphoreType.DMA,
        ],
    )
    def k(idx_hbm, table_hbm, scale_hbm, out_hbm, idx_all, acc_v, out_v,
          scale_v, sem_i, sem_g0, sem_g1, sem_o0, sem_o1):
        wid = lax.axis_index("s") * 2 + lax.axis_index("c")
        sem_g = (sem_g0, sem_g1)
        sem_o = (sem_o0, sem_o1)

        pltpu.sync_copy(scale_hbm, scale_v)
        scale = scale_v[...]

        descs = []
        for kk in range(K_PER_W):
            c = kk * NW + wid
            src = idx_hbm.at[pl.ds(c * ROWS_CHUNK, ROWS_CHUNK)]
            descs.append(pltpu.async_copy(
                src, idx_all.at[pl.ds(kk * ROWS_CHUNK, ROWS_CHUNK)], sem_i))
        for dsc in descs:
            dsc.wait()

        zero16 = jnp.zeros((LANES,), jnp.float32)

        def zero_acc(b):
            def t_body(t, tc):
                for g in range(D // LANES):
                    acc_v[b, t, pl.ds(g * LANES, LANES)] = zero16
                return tc
            lax.fori_loop(0, T_CHUNK, t_body, 0)

        def gathers(kk, b):
            return [
                pltpu.make_async_copy(
                    table_hbm.at[
                        idx_all.at[pl.ds(kk * ROWS_CHUNK + s * T_CHUNK,
                                         T_CHUNK)]],
                    acc_v.at[b],
                    sem_g[b])
                for s in range(N_SAMP)
            ]

        def fire_g(kk, b):
            for dsc in gathers(kk, b):
                dsc.start(add=True)

        def wait_g(kk, b):
            for dsc in gathers(kk, b):
                dsc.wait()

        def out_base(kk):
            return jnp.minimum((kk * NW + wid) * T_CHUNK, LAST_BASE)

        def out_desc(kk, b):
            return pltpu.make_async_copy(
                out_v.at[b], out_hbm.at[pl.ds(out_base(kk), T_CHUNK)],
                sem_o[b])

        def scale_and_rezero(b):
            def t_body(t, tc):
                for g in range(D // LANES):
                    sl = pl.ds(g * LANES, LANES)
                    out_v[b, t, sl] = acc_v[b, t, sl] * scale
                    acc_v[b, t, sl] = zero16
                return tc
            lax.fori_loop(0, T_CHUNK, t_body, 0)

        zero_acc(0)
        zero_acc(1)
        fire_g(0, 0)
        fire_g(1, 1)

        def body(i, carry):
            wait_g(2 * i, 0)

            @pl.when(i > 0)
            def _():
                out_desc(2 * i - 2, 0).wait()

            scale_and_rezero(0)
            out_desc(2 * i, 0).start()
            fire_g(2 * i + 2, 0)
            wait_g(2 * i + 1, 1)

            @pl.when(i > 0)
            def _():
                out_desc(2 * i - 1, 1).wait()

            scale_and_rezero(1)
            out_desc(2 * i + 1, 1).start()

            @pl.when(i < (K_PER_W - 3) // 2)
            def _():
                fire_g(2 * i + 3, 1)
            return carry

        lax.fori_loop(0, (K_PER_W - 1) // 2, body, 0)

        last = K_PER_W - 1
        wait_g(last, 0)
        out_desc(last - 2, 0).wait()
        scale_and_rezero(0)
        out_desc(last, 0).start()
        out_desc(last, 0).wait()
        out_desc(last - 1, 1).wait()

    return k(perm_idx, table, scale16)


def kernel(nodes, to_neighs, table, num_sample):
    del nodes
    bases = jnp.minimum(jnp.arange(N_CH) * T_CHUNK, LAST_BASE)
    tidx = bases[:, None] + jnp.arange(T_CHUNK)[None, :]
    perm = jnp.swapaxes(to_neighs.astype(jnp.int32)[tidx], 1, 2)
    perm_idx = perm.reshape(-1)
    ns = jnp.minimum(jnp.asarray(num_sample, jnp.float32),
                     jnp.float32(N_SAMP))
    scale16 = jnp.full((LANES,), 1.0, jnp.float32) / ns
    return _mean_agg(perm_idx, table, scale16)


# speedup vs baseline: 1.3139x; 1.0507x over previous
import functools

import jax
import jax.numpy as jnp
from jax import lax
from jax.experimental import pallas as pl
from jax.experimental.pallas import tpu as pltpu
from jax.experimental.pallas import tpu_sc as plsc

N_TGT = 50000
N_SAMP = 10
D = 128
LANES = 16
NW = 32
T_CHUNK = 64
ROWS_CHUNK = T_CHUNK * N_SAMP
LAST_BASE = N_TGT - T_CHUNK
K_PER_W = (-(-(-(-N_TGT // T_CHUNK)) // NW)) | 1
N_CH = NW * K_PER_W


def _mean_agg(perm_idx, table, scale16):
    mesh = plsc.VectorSubcoreMesh(core_axis_name="c", subcore_axis_name="s")

    @functools.partial(
        pl.kernel,
        mesh=mesh,
        out_type=jax.ShapeDtypeStruct((N_TGT, D), jnp.float32),
        scratch_types=[
            pltpu.VMEM((K_PER_W * ROWS_CHUNK,), jnp.int32),
            pltpu.VMEM((2, T_CHUNK, D), jnp.float32),
            pltpu.VMEM((2, T_CHUNK, D), jnp.float32),
            pltpu.VMEM((LANES,), jnp.float32),
            pltpu.SemaphoreType.DMA,
            pltpu.SemaphoreType.DMA,
            pltpu.SemaphoreType.DMA,
            pltpu.SemaphoreType.DMA,
            pltpu.SemaphoreType.DMA,
        ],
    )
    def k(idx_hbm, table_hbm, scale_hbm, out_hbm, idx_all, acc_v, out_v,
          scale_v, sem_i, sem_g0, sem_g1, sem_o0, sem_o1):
        wid = lax.axis_index("s") * 2 + lax.axis_index("c")
        sem_g = (sem_g0, sem_g1)
        sem_o = (sem_o0, sem_o1)

        pltpu.sync_copy(scale_hbm, scale_v)
        scale = scale_v[...]

        descs = []
        for kk in range(K_PER_W):
            c = kk * NW + wid
            src = idx_hbm.at[pl.ds(c * ROWS_CHUNK, ROWS_CHUNK)]
            descs.append(pltpu.async_copy(
                src, idx_all.at[pl.ds(kk * ROWS_CHUNK, ROWS_CHUNK)], sem_i))
        for dsc in descs:
            dsc.wait()

        zero16 = jnp.zeros((LANES,), jnp.float32)

        def zero_acc(b):
            def t_body(t, tc):
                for g in range(D // LANES):
                    acc_v[b, t, pl.ds(g * LANES, LANES)] = zero16
                return tc
            lax.fori_loop(0, T_CHUNK, t_body, 0)

        def gathers(kk, b):
            return [
                pltpu.make_async_copy(
                    table_hbm.at[
                        idx_all.at[pl.ds(kk * ROWS_CHUNK + s * T_CHUNK,
                                         T_CHUNK)]],
                    acc_v.at[b],
                    sem_g[b])
                for s in range(N_SAMP)
            ]

        def fire_g(kk, b):
            for dsc in gathers(kk, b):
                dsc.start(add=True)

        def wait_g(kk, b):
            for dsc in gathers(kk, b):
                dsc.wait()

        def out_base(kk):
            return jnp.minimum((kk * NW + wid) * T_CHUNK, LAST_BASE)

        def out_desc(kk, b):
            return pltpu.make_async_copy(
                out_v.at[b], out_hbm.at[pl.ds(out_base(kk), T_CHUNK)],
                sem_o[b])

        def scale_and_rezero(b):
            def t_body(t, tc):
                for g in range(D // LANES):
                    sl = pl.ds(g * LANES, LANES)
                    out_v[b, t, sl] = acc_v[b, t, sl] * scale
                    acc_v[b, t, sl] = zero16
                return tc
            lax.fori_loop(0, T_CHUNK, t_body, 0)

        zero_acc(0)
        zero_acc(1)
        fire_g(0, 0)
        fire_g(1, 1)

        def body(i, carry):
            wait_g(2 * i, 0)

            @pl.when(i > 0)
            def _():
                out_desc(2 * i - 2, 0).wait()

            scale_and_rezero(0)
            out_desc(2 * i, 0).start()
            fire_g(2 * i + 2, 0)
            wait_g(2 * i + 1, 1)

            @pl.when(i > 0)
            def _():
                out_desc(2 * i - 1, 1).wait()

            scale_and_rezero(1)
            out_desc(2 * i + 1, 1).start()

            @pl.when(i < (K_PER_W - 3) // 2)
            def _():
                fire_g(2 * i + 3, 1)
            return carry

        lax.fori_loop(0, (K_PER_W - 1) // 2, body, 0)

        last = K_PER_W - 1
        wait_g(last, 0)
        out_desc(last - 2, 0).wait()
        scale_and_rezero(0)
        out_desc(last, 0).start()
        out_desc(last, 0).wait()
        out_desc(last - 1, 1).wait()

    return k(perm_idx, table, scale16)


def kernel(nodes, to_neighs, table, num_sample):
    del nodes
    bases = jnp.minimum(jnp.arange(N_CH) * T_CHUNK, LAST_BASE)
    tidx = bases[:, None] + jnp.arange(T_CHUNK)[None, :]
    perm = jnp.swapaxes(to_neighs.astype(jnp.int32)[tidx], 1, 2)
    perm_idx = perm.reshape(-1)
    ns = jnp.minimum(jnp.asarray(num_sample, jnp.float32),
                     jnp.float32(N_SAMP))
    scale16 = jnp.full((LANES,), 1.0, jnp.float32) / ns
    return _mean_agg(perm_idx, table, scale16)


# speedup vs baseline: 1.3283x; 1.0110x over previous
import functools

import jax
import jax.numpy as jnp
from jax import lax
from jax.experimental import pallas as pl
from jax.experimental.pallas import tpu as pltpu
from jax.experimental.pallas import tpu_sc as plsc

N_TGT = 50000
N_SAMP = 10
D = 128
LANES = 16
NW = 32
T_CHUNK = 128
ROWS_CHUNK = T_CHUNK * N_SAMP
LAST_BASE = N_TGT - T_CHUNK
K_PER_W = (-(-(-(-N_TGT // T_CHUNK)) // NW)) | 1
N_CH = NW * K_PER_W


def _mean_agg(perm_idx, table, scale16):
    mesh = plsc.VectorSubcoreMesh(core_axis_name="c", subcore_axis_name="s")

    @functools.partial(
        pl.kernel,
        mesh=mesh,
        out_type=jax.ShapeDtypeStruct((N_TGT, D), jnp.float32),
        scratch_types=[
            pltpu.VMEM((K_PER_W * ROWS_CHUNK,), jnp.int32),
            pltpu.VMEM((2, T_CHUNK, D), jnp.float32),
            pltpu.VMEM((2, T_CHUNK, D), jnp.float32),
            pltpu.VMEM((LANES,), jnp.float32),
            pltpu.SemaphoreType.DMA,
            pltpu.SemaphoreType.DMA,
            pltpu.SemaphoreType.DMA,
            pltpu.SemaphoreType.DMA,
            pltpu.SemaphoreType.DMA,
        ],
    )
    def k(idx_hbm, table_hbm, scale_hbm, out_hbm, idx_all, acc_v, out_v,
          scale_v, sem_i, sem_g0, sem_g1, sem_o0, sem_o1):
        wid = lax.axis_index("s") * 2 + lax.axis_index("c")
        sem_g = (sem_g0, sem_g1)
        sem_o = (sem_o0, sem_o1)

        pltpu.sync_copy(scale_hbm, scale_v)
        scale = scale_v[...]

        descs = []
        for kk in range(K_PER_W):
            c = kk * NW + wid
            src = idx_hbm.at[pl.ds(c * ROWS_CHUNK, ROWS_CHUNK)]
            descs.append(pltpu.async_copy(
                src, idx_all.at[pl.ds(kk * ROWS_CHUNK, ROWS_CHUNK)], sem_i))
        for dsc in descs:
            dsc.wait()

        zero16 = jnp.zeros((LANES,), jnp.float32)

        def zero_acc(b):
            def t_body(t, tc):
                for g in range(D // LANES):
                    acc_v[b, t, pl.ds(g * LANES, LANES)] = zero16
                return tc
            lax.fori_loop(0, T_CHUNK, t_body, 0)

        def gathers(kk, b):
            return [
                pltpu.make_async_copy(
                    table_hbm.at[
                        idx_all.at[pl.ds(kk * ROWS_CHUNK + s * T_CHUNK,
                                         T_CHUNK)]],
                    acc_v.at[b],
                    sem_g[b])
                for s in range(N_SAMP)
            ]

        def fire_g(kk, b):
            for dsc in gathers(kk, b):
                dsc.start(add=True)

        def wait_g(kk, b):
            for dsc in gathers(kk, b):
                dsc.wait()

        def out_base(kk):
            return jnp.minimum((kk * NW + wid) * T_CHUNK, LAST_BASE)

        def out_desc(kk, b):
            return pltpu.make_async_copy(
                out_v.at[b], out_hbm.at[pl.ds(out_base(kk), T_CHUNK)],
                sem_o[b])

        def scale_and_rezero(b):
            def t_body(t, tc):
                for g in range(D // LANES):
                    sl = pl.ds(g * LANES, LANES)
                    out_v[b, t, sl] = acc_v[b, t, sl] * scale
                    acc_v[b, t, sl] = zero16
                return tc
            lax.fori_loop(0, T_CHUNK, t_body, 0)

        zero_acc(0)
        zero_acc(1)
        fire_g(0, 0)
        fire_g(1, 1)

        def body(i, carry):
            wait_g(2 * i, 0)

            @pl.when(i > 0)
            def _():
                out_desc(2 * i - 2, 0).wait()

            scale_and_rezero(0)
            out_desc(2 * i, 0).start()
            fire_g(2 * i + 2, 0)
            wait_g(2 * i + 1, 1)

            @pl.when(i > 0)
            def _():
                out_desc(2 * i - 1, 1).wait()

            scale_and_rezero(1)
            out_desc(2 * i + 1, 1).start()

            @pl.when(i < (K_PER_W - 3) // 2)
            def _():
                fire_g(2 * i + 3, 1)
            return carry

        lax.fori_loop(0, (K_PER_W - 1) // 2, body, 0)

        last = K_PER_W - 1
        wait_g(last, 0)
        out_desc(last - 2, 0).wait()
        scale_and_rezero(0)
        out_desc(last, 0).start()
        out_desc(last, 0).wait()
        out_desc(last - 1, 1).wait()

    return k(perm_idx, table, scale16)


def kernel(nodes, to_neighs, table, num_sample):
    del nodes
    bases = jnp.minimum(jnp.arange(N_CH) * T_CHUNK, LAST_BASE)
    tidx = bases[:, None] + jnp.arange(T_CHUNK)[None, :]
    perm = jnp.swapaxes(to_neighs.astype(jnp.int32)[tidx], 1, 2)
    perm_idx = perm.reshape(-1)
    ns = jnp.minimum(jnp.asarray(num_sample, jnp.float32),
                     jnp.float32(N_SAMP))
    scale16 = jnp.full((LANES,), 1.0, jnp.float32) / ns
    return _mean_agg(perm_idx, table, scale16)


# speedup vs baseline: 1.8089x; 1.3618x over previous
import functools

import jax
import jax.numpy as jnp
from jax import lax
from jax.experimental import pallas as pl
from jax.experimental.pallas import tpu as pltpu
from jax.experimental.pallas import tpu_sc as plsc

N_TGT = 50000
N_SAMP = 10
D = 128
LANES = 16
NW = 32
T_CHUNK = 128
ROWS_CHUNK = T_CHUNK * N_SAMP
LAST_BASE = N_TGT - T_CHUNK
K_PER_W = (-(-(-(-N_TGT // T_CHUNK)) // NW)) | 1
N_CH = NW * K_PER_W


def _mean_agg(perm_idx, table, scale16):
    mesh = plsc.VectorSubcoreMesh(core_axis_name="c", subcore_axis_name="s")

    @functools.partial(
        pl.kernel,
        mesh=mesh,
        out_type=jax.ShapeDtypeStruct((N_TGT, D), jnp.float32),
        scratch_types=[
            pltpu.VMEM((K_PER_W * ROWS_CHUNK,), jnp.int32),
            pltpu.VMEM((2, T_CHUNK, D), jnp.float32),
            pltpu.VMEM((2, T_CHUNK, D), jnp.float32),
            pltpu.VMEM((LANES,), jnp.float32),
            pltpu.SemaphoreType.DMA,
            pltpu.SemaphoreType.DMA,
            pltpu.SemaphoreType.DMA,
            pltpu.SemaphoreType.DMA,
            pltpu.SemaphoreType.DMA,
        ],
    )
    def k(idx_hbm, table_hbm, scale_hbm, out_hbm, idx_all, acc_v, out_v,
          scale_v, sem_i, sem_g0, sem_g1, sem_o0, sem_o1):
        wid = lax.axis_index("s") * 2 + lax.axis_index("c")
        sem_g = (sem_g0, sem_g1)
        sem_o = (sem_o0, sem_o1)

        pltpu.sync_copy(scale_hbm, scale_v)
        scale = scale_v[...]

        descs = []
        for kk in range(K_PER_W):
            base = jnp.minimum((kk * NW + wid) * T_CHUNK, LAST_BASE)
            for s in range(N_SAMP):
                src = idx_hbm.at[pl.ds(s * N_TGT + base, T_CHUNK)]
                descs.append(pltpu.async_copy(
                    src,
                    idx_all.at[pl.ds((kk * N_SAMP + s) * T_CHUNK, T_CHUNK)],
                    sem_i))
        for dsc in descs:
            dsc.wait()

        zero16 = jnp.zeros((LANES,), jnp.float32)

        def zero_acc(b):
            def t_body(t, tc):
                for g in range(D // LANES):
                    acc_v[b, t, pl.ds(g * LANES, LANES)] = zero16
                return tc
            lax.fori_loop(0, T_CHUNK, t_body, 0)

        def gathers(kk, b):
            return [
                pltpu.make_async_copy(
                    table_hbm.at[
                        idx_all.at[pl.ds(kk * ROWS_CHUNK + s * T_CHUNK,
                                         T_CHUNK)]],
                    acc_v.at[b],
                    sem_g[b])
                for s in range(N_SAMP)
            ]

        def fire_g(kk, b):
            for dsc in gathers(kk, b):
                dsc.start(add=True)

        def wait_g(kk, b):
            for dsc in gathers(kk, b):
                dsc.wait()

        def out_base(kk):
            return jnp.minimum((kk * NW + wid) * T_CHUNK, LAST_BASE)

        def out_desc(kk, b):
            return pltpu.make_async_copy(
                out_v.at[b], out_hbm.at[pl.ds(out_base(kk), T_CHUNK)],
                sem_o[b])

        def scale_and_rezero(b):
            def t_body(t, tc):
                for g in range(D // LANES):
                    sl = pl.ds(g * LANES, LANES)
                    out_v[b, t, sl] = acc_v[b, t, sl] * scale
                    acc_v[b, t, sl] = zero16
                return tc
            lax.fori_loop(0, T_CHUNK, t_body, 0)

        zero_acc(0)
        zero_acc(1)
        fire_g(0, 0)
        fire_g(1, 1)

        def body(i, carry):
            wait_g(2 * i, 0)

            @pl.when(i > 0)
            def _():
                out_desc(2 * i - 2, 0).wait()

            scale_and_rezero(0)
            out_desc(2 * i, 0).start()
            fire_g(2 * i + 2, 0)
            wait_g(2 * i + 1, 1)

            @pl.when(i > 0)
            def _():
                out_desc(2 * i - 1, 1).wait()

            scale_and_rezero(1)
            out_desc(2 * i + 1, 1).start()

            @pl.when(i < (K_PER_W - 3) // 2)
            def _():
                fire_g(2 * i + 3, 1)
            return carry

        lax.fori_loop(0, (K_PER_W - 1) // 2, body, 0)

        last = K_PER_W - 1
        wait_g(last, 0)
        out_desc(last - 2, 0).wait()
        scale_and_rezero(0)
        out_desc(last, 0).start()
        out_desc(last, 0).wait()
        out_desc(last - 1, 1).wait()

    return k(perm_idx, table, scale16)


def kernel(nodes, to_neighs, table, num_sample):
    del nodes
    perm_idx = jnp.swapaxes(to_neighs.astype(jnp.int32), 0, 1).reshape(-1)
    ns = jnp.minimum(jnp.asarray(num_sample, jnp.float32),
                     jnp.float32(N_SAMP))
    scale16 = jnp.full((LANES,), 1.0, jnp.float32) / ns
    return _mean_agg(perm_idx, table, scale16)


# speedup vs baseline: 1.8606x; 1.0286x over previous
import functools

import jax
import jax.numpy as jnp
from jax import lax
from jax.experimental import pallas as pl
from jax.experimental.pallas import tpu as pltpu
from jax.experimental.pallas import tpu_sc as plsc

N_TGT = 50000
N_SAMP = 10
D = 128
LANES = 16
NW = 32
T_CHUNK = 64
ROWS_CHUNK = T_CHUNK * N_SAMP
LAST_BASE = N_TGT - T_CHUNK
K_PER_W = (-(-(-(-N_TGT // T_CHUNK)) // NW)) | 1
N_CH = NW * K_PER_W


def _mean_agg(perm_idx, table, scale16):
    mesh = plsc.VectorSubcoreMesh(core_axis_name="c", subcore_axis_name="s")

    @functools.partial(
        pl.kernel,
        mesh=mesh,
        out_type=jax.ShapeDtypeStruct((N_TGT, D), jnp.float32),
        scratch_types=[
            pltpu.VMEM((K_PER_W * ROWS_CHUNK,), jnp.int32),
            pltpu.VMEM((2, T_CHUNK, D), jnp.float32),
            pltpu.VMEM((2, T_CHUNK, D), jnp.float32),
            pltpu.VMEM((LANES,), jnp.float32),
            pltpu.SemaphoreType.DMA,
            pltpu.SemaphoreType.DMA,
            pltpu.SemaphoreType.DMA,
            pltpu.SemaphoreType.DMA,
            pltpu.SemaphoreType.DMA,
        ],
    )
    def k(idx_hbm, table_hbm, scale_hbm, out_hbm, idx_all, acc_v, out_v,
          scale_v, sem_i, sem_g0, sem_g1, sem_o0, sem_o1):
        wid = lax.axis_index("s") * 2 + lax.axis_index("c")
        sem_g = (sem_g0, sem_g1)
        sem_o = (sem_o0, sem_o1)

        pltpu.sync_copy(scale_hbm, scale_v)
        scale = scale_v[...]

        descs = []
        for kk in range(K_PER_W):
            base = jnp.minimum((kk * NW + wid) * T_CHUNK, LAST_BASE)
            for s in range(N_SAMP):
                src = idx_hbm.at[pl.ds(s * N_TGT + base, T_CHUNK)]
                descs.append(pltpu.async_copy(
                    src,
                    idx_all.at[pl.ds((kk * N_SAMP + s) * T_CHUNK, T_CHUNK)],
                    sem_i))
        for dsc in descs:
            dsc.wait()

        zero16 = jnp.zeros((LANES,), jnp.float32)

        def zero_acc(b):
            def t_body(t, tc):
                for g in range(D // LANES):
                    acc_v[b, t, pl.ds(g * LANES, LANES)] = zero16
                return tc
            lax.fori_loop(0, T_CHUNK, t_body, 0)

        def gathers(kk, b):
            return [
                pltpu.make_async_copy(
                    table_hbm.at[
                        idx_all.at[pl.ds(kk * ROWS_CHUNK + s * T_CHUNK,
                                         T_CHUNK)]],
                    acc_v.at[b],
                    sem_g[b])
                for s in range(N_SAMP)
            ]

        def fire_g(kk, b):
            for dsc in gathers(kk, b):
                dsc.start(add=True)

        def wait_g(kk, b):
            for dsc in gathers(kk, b):
                dsc.wait()

        def out_base(kk):
            return jnp.minimum((kk * NW + wid) * T_CHUNK, LAST_BASE)

        def out_desc(kk, b):
            return pltpu.make_async_copy(
                out_v.at[b], out_hbm.at[pl.ds(out_base(kk), T_CHUNK)],
                sem_o[b])

        def scale_and_rezero(b):
            def t_body(t, tc):
                for g in range(D // LANES):
                    sl = pl.ds(g * LANES, LANES)
                    out_v[b, t, sl] = acc_v[b, t, sl] * scale
                    acc_v[b, t, sl] = zero16
                return tc
            lax.fori_loop(0, T_CHUNK, t_body, 0)

        zero_acc(0)
        zero_acc(1)
        fire_g(0, 0)
        fire_g(1, 1)

        def body(i, carry):
            wait_g(2 * i, 0)

            @pl.when(i > 0)
            def _():
                out_desc(2 * i - 2, 0).wait()

            scale_and_rezero(0)
            out_desc(2 * i, 0).start()
            fire_g(2 * i + 2, 0)
            wait_g(2 * i + 1, 1)

            @pl.when(i > 0)
            def _():
                out_desc(2 * i - 1, 1).wait()

            scale_and_rezero(1)
            out_desc(2 * i + 1, 1).start()

            @pl.when(i < (K_PER_W - 3) // 2)
            def _():
                fire_g(2 * i + 3, 1)
            return carry

        lax.fori_loop(0, (K_PER_W - 1) // 2, body, 0)

        last = K_PER_W - 1
        wait_g(last, 0)
        out_desc(last - 2, 0).wait()
        scale_and_rezero(0)
        out_desc(last, 0).start()
        out_desc(last, 0).wait()
        out_desc(last - 1, 1).wait()

    return k(perm_idx, table, scale16)


def kernel(nodes, to_neighs, table, num_sample):
    del nodes
    perm_idx = jnp.swapaxes(to_neighs.astype(jnp.int32), 0, 1).reshape(-1)
    ns = jnp.minimum(jnp.asarray(num_sample, jnp.float32),
                     jnp.float32(N_SAMP))
    scale16 = jnp.full((LANES,), 1.0, jnp.float32) / ns
    return _mean_agg(perm_idx, table, scale16)
